# baseline (speedup 1.0000x reference)
import dataclasses
import functools

import jax
import jax.numpy as jnp
from jax import lax
from jax.experimental import pallas as pl
from jax.experimental.pallas import tpu as pltpu
from jax.experimental.pallas import tpu_sc as plsc

NC = 2
NS = 16
NW = NC * NS
L = 16
UNIT = 128
ROW = 8

_CP = pltpu.CompilerParams(use_tc_tiling_on_sc=False)
if "needs_layout_passes" in pltpu.CompilerParams.__dataclass_fields__:
    _CP = dataclasses.replace(_CP, needs_layout_passes=False)

_MESH = plsc.VectorSubcoreMesh(core_axis_name="c", subcore_axis_name="s")


def _iota():
    return lax.iota(jnp.int32, L)


def _cvec(c):
    return jnp.full((L,), c, jnp.int32)


def _fvec(x):
    return jnp.full((L,), x, jnp.float32)


def _rsqrt(x):
    i = plsc.bitcast(x, jnp.int32)
    i = jnp.full((L,), 0x5F3759DF, jnp.int32) - lax.shift_right_logical(
        i, jnp.full((L,), 1, jnp.int32))
    y = plsc.bitcast(i, jnp.float32)
    h = x * _fvec(0.5)
    for _ in range(3):
        y = y * (_fvec(1.5) - h * y * y)
    return y


def _accumulate_kernel(V, Vp, Fp):
    FW = Fp // NW
    N = FW // UNIT
    assert N % 4 == 0 and N >= 8
    K = (N - 4) // 4
    ZR = Vp // NS

    @functools.partial(
        pl.kernel,
        mesh=_MESH,
        out_type=jax.ShapeDtypeStruct((NC, Vp, ROW), jnp.float32),
        scratch_types=[
            pltpu.VMEM_SHARED((Vp, ROW), jnp.float32),
            pltpu.VMEM((4, 6, UNIT), jnp.int32),
            pltpu.VMEM((UNIT, ROW), jnp.float32),
            pltpu.VMEM((UNIT, ROW), jnp.float32),
            pltpu.VMEM((UNIT, ROW), jnp.float32),
            pltpu.VMEM((UNIT, ROW), jnp.float32),
            pltpu.VMEM((UNIT, ROW), jnp.float32),
            pltpu.VMEM((UNIT, ROW), jnp.float32),
            pltpu.VMEM((UNIT, ROW), jnp.float32),
            pltpu.VMEM((UNIT, ROW), jnp.float32),
            pltpu.VMEM((UNIT, ROW), jnp.float32),
            pltpu.VMEM((UNIT, ROW), jnp.float32),
            pltpu.VMEM((UNIT, ROW), jnp.float32),
            pltpu.VMEM((UNIT, ROW), jnp.float32),
            pltpu.VMEM((UNIT, ROW), jnp.float32),
            pltpu.VMEM((UNIT, ROW), jnp.float32),
            pltpu.SemaphoreType.DMA,
            pltpu.SemaphoreType.DMA,
            pltpu.SemaphoreType.DMA,
            pltpu.SemaphoreType.DMA,
            pltpu.SemaphoreType.DMA,
            pltpu.SemaphoreType.DMA,
            pltpu.SemaphoreType.DMA,
            pltpu.SemaphoreType.DMA,
        ],
        compiler_params=_CP,
    )
    def k(table_h, fidx_h, zero_h, out_h, acc, ib,
          pa0, pa1, pa2, pa3, pa4, pa5, pb0, pb1, pb2, pb3, pb4, pb5,
          res0, res1, si0, si1, si2, si3, sg0, sg1, ss0, ss1):
        pb = [[pa0, pa1, pa2, pa3, pa4, pa5], [pb0, pb1, pb2, pb3, pb4, pb5]]
        res = [res0, res1]
        sem_i = [si0, si1, si2, si3]
        sem_g = [sg0, sg1]
        sem_s = [ss0, ss1]
        cid = lax.axis_index("c")
        sid = lax.axis_index("s")
        wid = cid * NS + sid
        pltpu.sync_copy(zero_h, acc.at[pl.ds(sid * ZR, ZR), :])
        pltpu.sync_copy(zero_h.at[pl.ds(0, UNIT), :], res0)
        pltpu.sync_copy(zero_h.at[pl.ds(0, UNIT), :], res1)
        plsc.subcore_barrier()

        base_u = wid * N
        iota = _iota()

        def fire_idx(u, r):
            pltpu.async_copy(fidx_h.at[base_u + u], ib.at[r], sem_i[r])

        def wait_idx(u, r):
            pltpu.make_async_copy(fidx_h.at[base_u + u], ib.at[r],
                                  sem_i[r]).wait()

        def fire_gath(r, p):
            for j in range(6):
                pltpu.async_copy(table_h.at[ib.at[r, j]], pb[p][j], sem_g[p])

        def wait_gath(r, p):
            for j in range(6):
                pltpu.make_async_copy(table_h.at[ib.at[r, j]], pb[p][j],
                                      sem_g[p]).wait()

        def fire_scat(r, p):
            for j in range(3):
                pltpu.async_copy(res[p], acc.at[ib.at[r, j]], sem_s[p],
                                 add=True)

        def wait_scat(r, p):
            for j in range(3):
                pltpu.make_async_copy(res[p], acc.at[ib.at[r, j]],
                                      sem_s[p]).wait()

        def compute(p):
            for g in range(UNIT // L):
                rows = iota + _cvec(g * L)

                def ld(ref, c):
                    return plsc.load_gather(ref, [rows, _cvec(c)])

                b0, b1, b2, b3, b4, b5 = pb[p]
                p0x, p0y, p0z = ld(b0, 0), ld(b0, 1), ld(b0, 2)
                p1x, p1y, p1z = ld(b1, 0), ld(b1, 1), ld(b1, 2)
                p2x, p2y, p2z = ld(b2, 0), ld(b2, 1), ld(b2, 2)
                t0u, t0v = ld(b3, 0), ld(b3, 1)
                t1u, t1v = ld(b4, 0), ld(b4, 1)
                t2u, t2v = ld(b5, 0), ld(b5, 1)
                e1x, e1y, e1z = p1x - p0x, p1y - p0y, p1z - p0z
                e2x, e2y, e2z = p2x - p0x, p2y - p0y, p2z - p0z
                nx = e1y * e2z - e1z * e2y
                ny = e1z * e2x - e1x * e2z
                nz = e1x * e2y - e1y * e2x
                u1, v1 = t1u - t0u, t1v - t0v
                u2, v2 = t2u - t0u, t2v - t0v
                den = u1 * v2 - v1 * u2
                den_safe = jnp.where(den > _fvec(0.0),
                                     jnp.maximum(den, _fvec(1e-6)),
                                     jnp.minimum(den, _fvec(-1e-6)))
                inv = _fvec(1.0) / den_safe
                tgx = (e1x * v2 - e2x * v1) * inv
                tgy = (e1y * v2 - e2y * v1) * inv
                tgz = (e1z * v2 - e2z * v1) * inv
                rr = res[p]
                plsc.store_scatter(rr, [rows, _cvec(0)], nx)
                plsc.store_scatter(rr, [rows, _cvec(1)], ny)
                plsc.store_scatter(rr, [rows, _cvec(2)], nz)
                plsc.store_scatter(rr, [rows, _cvec(3)], tgx)
                plsc.store_scatter(rr, [rows, _cvec(4)], tgy)
                plsc.store_scatter(rr, [rows, _cvec(5)], tgz)

        for u in range(4):
            fire_idx(u, u)
        wait_idx(0, 0)
        fire_gath(0, 0)
        wait_gath(0, 0)
        wait_idx(1, 1)
        fire_gath(1, 1)
        compute(0)
        fire_scat(0, 0)
        wait_gath(1, 1)
        wait_idx(2, 2)
        fire_gath(2, 0)
        compute(1)
        fire_scat(1, 1)

        @pl.loop(0, K)
        def _(kk):
            for d in range(4):
                u = 2 + 4 * kk + d
                r_u = (2 + d) % 4
                r_n1 = (3 + d) % 4
                r_n2 = d % 4
                p = d % 2
                q = 1 - p
                wait_gath(r_u, p)
                wait_scat(r_n2, p)
                fire_idx(u + 2, r_n2)
                wait_idx(u + 1, r_n1)
                fire_gath(r_n1, q)
                compute(p)
                fire_scat(r_u, p)

        wait_gath(2, 0)
        wait_scat(0, 0)
        wait_idx(N - 1, 3)
        fire_gath(3, 1)
        compute(0)
        fire_scat(2, 0)
        wait_gath(3, 1)
        wait_scat(1, 1)
        compute(1)
        fire_scat(3, 1)
        wait_scat(2, 0)
        wait_scat(3, 1)

        plsc.subcore_barrier()
        pltpu.sync_copy(acc.at[pl.ds(sid * ZR, ZR), :],
                        out_h.at[cid, pl.ds(sid * ZR, ZR), :])

    return k


def _finalize_kernel(Vp):
    WV = Vp // NW

    @functools.partial(
        pl.kernel,
        mesh=_MESH,
        out_type=jax.ShapeDtypeStruct((2, Vp, 4), jnp.float32),
        scratch_types=[
            pltpu.VMEM((WV, ROW), jnp.float32),
            pltpu.VMEM((WV, ROW), jnp.float32),
            pltpu.VMEM((WV, 4), jnp.float32),
            pltpu.VMEM((WV, 4), jnp.float32),
        ],
        compiler_params=_CP,
    )
    def k(in_h, out_h, a0, a1, nout, tout):
        cid = lax.axis_index("c")
        sid = lax.axis_index("s")
        wid = cid * NS + sid
        b = wid * WV
        pltpu.sync_copy(in_h.at[0, pl.ds(b, WV), :], a0)
        pltpu.sync_copy(in_h.at[1, pl.ds(b, WV), :], a1)
        iota = _iota()

        @pl.loop(0, WV // L)
        def _(g):
            rows = iota + g * L

            def ld(c):
                cc = _cvec(c)
                return (plsc.load_gather(a0, [rows, cc]) +
                        plsc.load_gather(a1, [rows, cc]))

            nx, ny, nz = ld(0), ld(1), ld(2)
            tx, ty, tz = ld(3), ld(4), ld(5)
            d = nx * nx + ny * ny + nz * nz
            cond = d > _fvec(1e-20)
            zero = _fvec(0.0)
            nx = jnp.where(cond, nx, zero)
            ny = jnp.where(cond, ny, zero)
            nz = jnp.where(cond, nz, _fvec(1.0))
            dsel = jnp.where(cond, d, _fvec(1.0))
            r = _rsqrt(jnp.maximum(dsel, _fvec(1e-20)))
            onx, ony, onz = nx * r, ny * r, nz * r
            dt = tx * tx + ty * ty + tz * tz
            rt = _rsqrt(jnp.maximum(dt, _fvec(1e-20)))
            ttx, tty, ttz = tx * rt, ty * rt, tz * rt
            dtn = ttx * onx + tty * ony + ttz * onz
            wx = ttx - dtn * onx
            wy = tty - dtn * ony
            wz = ttz - dtn * onz
            dw = wx * wx + wy * wy + wz * wz
            rw = _rsqrt(jnp.maximum(dw, _fvec(1e-20)))
            plsc.store_scatter(nout, [rows, _cvec(0)], onx)
            plsc.store_scatter(nout, [rows, _cvec(1)], ony)
            plsc.store_scatter(nout, [rows, _cvec(2)], onz)
            plsc.store_scatter(tout, [rows, _cvec(0)], wx * rw)
            plsc.store_scatter(tout, [rows, _cvec(1)], wy * rw)
            plsc.store_scatter(tout, [rows, _cvec(2)], wz * rw)

        pltpu.sync_copy(nout, out_h.at[0, pl.ds(b, WV), :])
        pltpu.sync_copy(tout, out_h.at[1, pl.ds(b, WV), :])

    return k


def kernel(positions, texcoords, faces, uv_faces):
    V = positions.shape[0]
    F = faces.shape[0]
    per_w = -(-F // (NW * 4 * UNIT)) * (4 * UNIT)
    Fp = per_w * NW
    Vp = -(-V // (NW * L)) * (NW * L)

    table = jnp.zeros((2 * V, ROW), jnp.float32)
    table = table.at[:V, :3].set(positions.astype(jnp.float32))
    table = table.at[V:, :2].set(texcoords.astype(jnp.float32))
    f_t = faces.astype(jnp.int32).T
    u_t = uv_faces.astype(jnp.int32).T + V
    fidx = jnp.concatenate([f_t, u_t], axis=0)
    fidx = jnp.pad(fidx, ((0, 0), (0, Fp - F)))
    fidx = fidx.reshape(6, Fp // UNIT, UNIT).transpose(1, 0, 2)
    zero = jnp.zeros((Vp // NS, ROW), jnp.float32)

    partial = _accumulate_kernel(V, Vp, Fp)(table, fidx, zero)
    out2 = _finalize_kernel(Vp)(partial)
    return jnp.concatenate([out2[0, :V, :3], out2[1, :V, :3]], axis=0)


# speedup vs baseline: 1.3070x; 1.3070x over previous
import dataclasses
import functools

import jax
import jax.numpy as jnp
from jax import lax
from jax.experimental import pallas as pl
from jax.experimental.pallas import tpu as pltpu
from jax.experimental.pallas import tpu_sc as plsc

NC = 2
NS = 16
NW = NC * NS
L = 16
UNIT = 80
ROW = 8

_CP = pltpu.CompilerParams(use_tc_tiling_on_sc=False)
if "needs_layout_passes" in pltpu.CompilerParams.__dataclass_fields__:
    _CP = dataclasses.replace(_CP, needs_layout_passes=False)

_MESH = plsc.VectorSubcoreMesh(core_axis_name="c", subcore_axis_name="s")


def _iota():
    return lax.iota(jnp.int32, L)


def _cvec(c):
    return jnp.full((L,), c, jnp.int32)


def _fvec(x):
    return jnp.full((L,), x, jnp.float32)


def _rsqrt(x):
    i = plsc.bitcast(x, jnp.int32)
    i = jnp.full((L,), 0x5F3759DF, jnp.int32) - lax.shift_right_logical(
        i, jnp.full((L,), 1, jnp.int32))
    y = plsc.bitcast(i, jnp.float32)
    h = x * _fvec(0.5)
    for _ in range(3):
        y = y * (_fvec(1.5) - h * y * y)
    return y


def _accumulate_kernel(V, Vp, F):
    per_w = -(-F // (NW * 4 * UNIT)) * (4 * UNIT)
    N = per_w // UNIT
    assert N % 4 == 0 and N >= 8 and F % UNIT == 0
    K = (N - 4) // 4
    ZR = Vp // NS
    G = UNIT // L

    @functools.partial(
        pl.kernel,
        mesh=_MESH,
        out_type=jax.ShapeDtypeStruct((NC, Vp, ROW), jnp.float32),
        scratch_types=[
            pltpu.VMEM_SHARED((Vp, ROW), jnp.float32),
            pltpu.VMEM((4, 2, UNIT, 3), jnp.int32),
            pltpu.VMEM((4, 6, UNIT), jnp.int32),
            pltpu.VMEM((UNIT, ROW), jnp.float32),
            pltpu.VMEM((UNIT, ROW), jnp.float32),
            pltpu.VMEM((UNIT, ROW), jnp.float32),
            pltpu.VMEM((UNIT, ROW), jnp.float32),
            pltpu.VMEM((UNIT, ROW), jnp.float32),
            pltpu.VMEM((UNIT, ROW), jnp.float32),
            pltpu.VMEM((UNIT, ROW), jnp.float32),
            pltpu.VMEM((UNIT, ROW), jnp.float32),
            pltpu.VMEM((UNIT, ROW), jnp.float32),
            pltpu.VMEM((UNIT, ROW), jnp.float32),
            pltpu.VMEM((UNIT, ROW), jnp.float32),
            pltpu.VMEM((UNIT, ROW), jnp.float32),
            pltpu.VMEM((UNIT, ROW), jnp.float32),
            pltpu.VMEM((UNIT, ROW), jnp.float32),
            pltpu.SemaphoreType.DMA,
            pltpu.SemaphoreType.DMA,
            pltpu.SemaphoreType.DMA,
            pltpu.SemaphoreType.DMA,
            pltpu.SemaphoreType.DMA,
            pltpu.SemaphoreType.DMA,
            pltpu.SemaphoreType.DMA,
            pltpu.SemaphoreType.DMA,
        ],
        compiler_params=_CP,
    )
    def k(pos_h, tex_h, fac_h, uvf_h, zero_h, out_h, acc, rb, ib,
          qa0, qa1, qa2, qb0, qb1, qb2, ta0, ta1, ta2, tb0, tb1, tb2,
          res0, res1, si0, si1, si2, si3, sg0, sg1, ss0, ss1):
        qp = [[qa0, qa1, qa2], [qb0, qb1, qb2]]
        qt = [[ta0, ta1, ta2], [tb0, tb1, tb2]]
        res = [res0, res1]
        sem_i = [si0, si1, si2, si3]
        sem_g = [sg0, sg1]
        sem_s = [ss0, ss1]
        cid = lax.axis_index("c")
        sid = lax.axis_index("s")
        wid = cid * NS + sid
        pltpu.sync_copy(zero_h, acc.at[pl.ds(sid * ZR, ZR), :])
        pltpu.sync_copy(zero_h.at[pl.ds(0, UNIT), :], res0)
        pltpu.sync_copy(zero_h.at[pl.ds(0, UNIT), :], res1)
        plsc.subcore_barrier()

        base_f = wid * per_w
        iota = _iota()

        def unit_base(u):
            return jnp.minimum(base_f + u * UNIT, F - UNIT)

        def fire_idx(u, r):
            b = unit_base(u)
            pltpu.async_copy(fac_h.at[pl.ds(b, UNIT), :], rb.at[r, 0],
                             sem_i[r])
            pltpu.async_copy(uvf_h.at[pl.ds(b, UNIT), :], rb.at[r, 1],
                             sem_i[r])

        def wait_idx(u, r):
            b = unit_base(u)
            pltpu.make_async_copy(fac_h.at[pl.ds(b, UNIT), :], rb.at[r, 0],
                                  sem_i[r]).wait()
            pltpu.make_async_copy(uvf_h.at[pl.ds(b, UNIT), :], rb.at[r, 1],
                                  sem_i[r]).wait()

        def transpose_idx(r):
            for g in range(G):
                rows = iota + _cvec(g * L)
                for c in range(3):
                    ib[r, c, pl.ds(g * L, L)] = plsc.load_gather(
                        rb.at[r, 0], [rows, _cvec(c)])
                    ib[r, 3 + c, pl.ds(g * L, L)] = plsc.load_gather(
                        rb.at[r, 1], [rows, _cvec(c)])

        def fire_gath(r, p):
            for c in range(3):
                pltpu.async_copy(pos_h.at[ib.at[r, c]], qp[p][c], sem_g[p])
                pltpu.async_copy(tex_h.at[ib.at[r, 3 + c]], qt[p][c],
                                 sem_g[p])

        def wait_gath(r, p):
            for c in range(3):
                pltpu.make_async_copy(pos_h.at[ib.at[r, c]], qp[p][c],
                                      sem_g[p]).wait()
                pltpu.make_async_copy(tex_h.at[ib.at[r, 3 + c]], qt[p][c],
                                      sem_g[p]).wait()

        def fire_scat(r, p):
            for j in range(3):
                pltpu.async_copy(res[p], acc.at[ib.at[r, j]], sem_s[p],
                                 add=True)

        def wait_scat(r, p):
            for j in range(3):
                pltpu.make_async_copy(res[p], acc.at[ib.at[r, j]],
                                      sem_s[p]).wait()

        def compute(u, p):
            sc = jnp.where(base_f + u * UNIT < F, 1.0, 0.0)
            svec = jnp.broadcast_to(sc.astype(jnp.float32), (L,))
            b0, b1, b2 = qp[p]
            c0, c1, c2 = qt[p]
            rr = res[p]
            for g in range(G):
                rows = iota + _cvec(g * L)

                def ld(ref, c):
                    return plsc.load_gather(ref, [rows, _cvec(c)])

                p0x, p0y, p0z = ld(b0, 0), ld(b0, 1), ld(b0, 2)
                p1x, p1y, p1z = ld(b1, 0), ld(b1, 1), ld(b1, 2)
                p2x, p2y, p2z = ld(b2, 0), ld(b2, 1), ld(b2, 2)
                t0u, t0v = ld(c0, 0), ld(c0, 1)
                t1u, t1v = ld(c1, 0), ld(c1, 1)
                t2u, t2v = ld(c2, 0), ld(c2, 1)
                e1x, e1y, e1z = p1x - p0x, p1y - p0y, p1z - p0z
                e2x, e2y, e2z = p2x - p0x, p2y - p0y, p2z - p0z
                nx = e1y * e2z - e1z * e2y
                ny = e1z * e2x - e1x * e2z
                nz = e1x * e2y - e1y * e2x
                u1, v1 = t1u - t0u, t1v - t0v
                u2, v2 = t2u - t0u, t2v - t0v
                den = u1 * v2 - v1 * u2
                den_safe = jnp.where(den > _fvec(0.0),
                                     jnp.maximum(den, _fvec(1e-6)),
                                     jnp.minimum(den, _fvec(-1e-6)))
                inv = svec / den_safe
                tgx = (e1x * v2 - e2x * v1) * inv
                tgy = (e1y * v2 - e2y * v1) * inv
                tgz = (e1z * v2 - e2z * v1) * inv
                plsc.store_scatter(rr, [rows, _cvec(0)], nx * svec)
                plsc.store_scatter(rr, [rows, _cvec(1)], ny * svec)
                plsc.store_scatter(rr, [rows, _cvec(2)], nz * svec)
                plsc.store_scatter(rr, [rows, _cvec(3)], tgx)
                plsc.store_scatter(rr, [rows, _cvec(4)], tgy)
                plsc.store_scatter(rr, [rows, _cvec(5)], tgz)

        for u in range(4):
            fire_idx(u, u)
        wait_idx(0, 0)
        transpose_idx(0)
        fire_gath(0, 0)
        wait_gath(0, 0)
        wait_idx(1, 1)
        transpose_idx(1)
        fire_gath(1, 1)
        compute(0, 0)
        fire_scat(0, 0)
        wait_gath(1, 1)
        wait_idx(2, 2)
        transpose_idx(2)
        fire_gath(2, 0)
        compute(1, 1)
        fire_scat(1, 1)

        @pl.loop(0, K)
        def _(kk):
            for d in range(4):
                u = 2 + 4 * kk + d
                r_u = (2 + d) % 4
                r_n1 = (3 + d) % 4
                r_n2 = d % 4
                p = d % 2
                q = 1 - p
                wait_gath(r_u, p)
                wait_scat(r_n2, p)
                fire_idx(u + 2, r_n2)
                wait_idx(u + 1, r_n1)
                transpose_idx(r_n1)
                fire_gath(r_n1, q)
                compute(u, p)
                fire_scat(r_u, p)

        wait_gath(2, 0)
        wait_scat(0, 0)
        wait_idx(N - 1, 3)
        transpose_idx(3)
        fire_gath(3, 1)
        compute(N - 2, 0)
        fire_scat(2, 0)
        wait_gath(3, 1)
        wait_scat(1, 1)
        compute(N - 1, 1)
        fire_scat(3, 1)
        wait_scat(2, 0)
        wait_scat(3, 1)

        plsc.subcore_barrier()
        pltpu.sync_copy(acc.at[pl.ds(sid * ZR, ZR), :],
                        out_h.at[cid, pl.ds(sid * ZR, ZR), :])

    return k


def _finalize_kernel(V, Vp):
    WV = Vp // NW
    LASTW = V - (NW - 1) * WV
    assert 0 < LASTW <= WV and LASTW % 8 == 0

    @functools.partial(
        pl.kernel,
        mesh=_MESH,
        out_type=jax.ShapeDtypeStruct((2 * V, 3), jnp.float32),
        scratch_types=[
            pltpu.VMEM((WV, ROW), jnp.float32),
            pltpu.VMEM((WV, ROW), jnp.float32),
            pltpu.VMEM((WV, 3), jnp.float32),
            pltpu.VMEM((WV, 3), jnp.float32),
        ],
        compiler_params=_CP,
    )
    def k(in_h, out_h, a0, a1, nout, tout):
        cid = lax.axis_index("c")
        sid = lax.axis_index("s")
        wid = cid * NS + sid
        b = wid * WV
        pltpu.sync_copy(in_h.at[0, pl.ds(b, WV), :], a0)
        pltpu.sync_copy(in_h.at[1, pl.ds(b, WV), :], a1)
        iota = _iota()

        @pl.loop(0, WV // L)
        def _(g):
            rows = iota + g * L

            def ld(c):
                cc = _cvec(c)
                return (plsc.load_gather(a0, [rows, cc]) +
                        plsc.load_gather(a1, [rows, cc]))

            nx, ny, nz = ld(0), ld(1), ld(2)
            tx, ty, tz = ld(3), ld(4), ld(5)
            d = nx * nx + ny * ny + nz * nz
            cond = d > _fvec(1e-20)
            zero = _fvec(0.0)
            nx = jnp.where(cond, nx, zero)
            ny = jnp.where(cond, ny, zero)
            nz = jnp.where(cond, nz, _fvec(1.0))
            dsel = jnp.where(cond, d, _fvec(1.0))
            r = _rsqrt(jnp.maximum(dsel, _fvec(1e-20)))
            onx, ony, onz = nx * r, ny * r, nz * r
            dt = tx * tx + ty * ty + tz * tz
            rt = _rsqrt(jnp.maximum(dt, _fvec(1e-20)))
            ttx, tty, ttz = tx * rt, ty * rt, tz * rt
            dtn = ttx * onx + tty * ony + ttz * onz
            wx = ttx - dtn * onx
            wy = tty - dtn * ony
            wz = ttz - dtn * onz
            dw = wx * wx + wy * wy + wz * wz
            rw = _rsqrt(jnp.maximum(dw, _fvec(1e-20)))
            plsc.store_scatter(nout, [rows, _cvec(0)], onx)
            plsc.store_scatter(nout, [rows, _cvec(1)], ony)
            plsc.store_scatter(nout, [rows, _cvec(2)], onz)
            plsc.store_scatter(tout, [rows, _cvec(0)], wx * rw)
            plsc.store_scatter(tout, [rows, _cvec(1)], wy * rw)
            plsc.store_scatter(tout, [rows, _cvec(2)], wz * rw)

        @pl.when(wid < NW - 1)
        def _():
            pltpu.sync_copy(nout, out_h.at[pl.ds(b, WV), :])
            pltpu.sync_copy(tout, out_h.at[pl.ds(V + b, WV), :])

        @pl.when(wid == NW - 1)
        def _():
            pltpu.sync_copy(nout.at[pl.ds(0, LASTW), :],
                            out_h.at[pl.ds(b, LASTW), :])
            pltpu.sync_copy(tout.at[pl.ds(0, LASTW), :],
                            out_h.at[pl.ds(V + b, LASTW), :])

    return k


def kernel(positions, texcoords, faces, uv_faces):
    V = positions.shape[0]
    F = faces.shape[0]
    Vp = -(-V // (NW * L)) * (NW * L)
    zero = jnp.zeros((Vp // NS, ROW), jnp.float32)

    pos8 = jnp.pad(positions.astype(jnp.float32), ((0, 0), (0, ROW - 3)))
    tex8 = jnp.pad(texcoords.astype(jnp.float32), ((0, 0), (0, ROW - 2)))
    partial = _accumulate_kernel(V, Vp, F)(
        pos8, tex8, faces.astype(jnp.int32), uv_faces.astype(jnp.int32), zero)
    return _finalize_kernel(V, Vp)(partial)


# speedup vs baseline: 4.0603x; 3.1067x over previous
import dataclasses
import functools

import jax
import jax.numpy as jnp
from jax import lax
from jax.experimental import pallas as pl
from jax.experimental.pallas import tpu as pltpu
from jax.experimental.pallas import tpu_sc as plsc

NC = 2
NS = 16
NW = NC * NS
L = 16
UNIT = 80
ROW = 8

_CP = pltpu.CompilerParams(use_tc_tiling_on_sc=False)
if "needs_layout_passes" in pltpu.CompilerParams.__dataclass_fields__:
    _CP = dataclasses.replace(_CP, needs_layout_passes=False)

_MESH = plsc.VectorSubcoreMesh(core_axis_name="c", subcore_axis_name="s")


def _iota():
    return lax.iota(jnp.int32, L)


def _cvec(c):
    return jnp.full((L,), c, jnp.int32)


def _fvec(x):
    return jnp.full((L,), x, jnp.float32)


def _rsqrt(x):
    i = plsc.bitcast(x, jnp.int32)
    i = jnp.full((L,), 0x5F3759DF, jnp.int32) - lax.shift_right_logical(
        i, jnp.full((L,), 1, jnp.int32))
    y = plsc.bitcast(i, jnp.float32)
    h = x * _fvec(0.5)
    for _ in range(3):
        y = y * (_fvec(1.5) - h * y * y)
    return y


def _accumulate_kernel(V, Vp, F):
    per_w = -(-F // (NW * 4 * UNIT)) * (4 * UNIT)
    N = per_w // UNIT
    assert N % 4 == 0 and N >= 8 and F % UNIT == 0
    K = (N - 4) // 4
    ZR = Vp // NS
    G = UNIT // L

    @functools.partial(
        pl.kernel,
        mesh=_MESH,
        out_type=jax.ShapeDtypeStruct((NC, Vp, ROW), jnp.float32),
        scratch_types=[
            pltpu.VMEM_SHARED((Vp, ROW), jnp.float32),
            pltpu.VMEM((4, 6, UNIT), jnp.int32),
            pltpu.VMEM((UNIT, ROW), jnp.float32),
            pltpu.VMEM((UNIT, ROW), jnp.float32),
            pltpu.VMEM((UNIT, ROW), jnp.float32),
            pltpu.VMEM((UNIT, ROW), jnp.float32),
            pltpu.VMEM((UNIT, ROW), jnp.float32),
            pltpu.VMEM((UNIT, ROW), jnp.float32),
            pltpu.VMEM((UNIT, ROW), jnp.float32),
            pltpu.VMEM((UNIT, ROW), jnp.float32),
            pltpu.VMEM((UNIT, ROW), jnp.float32),
            pltpu.VMEM((UNIT, ROW), jnp.float32),
            pltpu.VMEM((UNIT, ROW), jnp.float32),
            pltpu.VMEM((UNIT, ROW), jnp.float32),
            pltpu.VMEM((UNIT, ROW), jnp.float32),
            pltpu.VMEM((UNIT, ROW), jnp.float32),
            pltpu.SemaphoreType.DMA,
            pltpu.SemaphoreType.DMA,
            pltpu.SemaphoreType.DMA,
            pltpu.SemaphoreType.DMA,
            pltpu.SemaphoreType.DMA,
            pltpu.SemaphoreType.DMA,
            pltpu.SemaphoreType.DMA,
            pltpu.SemaphoreType.DMA,
        ],
        compiler_params=_CP,
    )
    def k(table_h, fidx_h, zero_h, out_h, acc, ib,
          qa0, qa1, qa2, qb0, qb1, qb2, ta0, ta1, ta2, tb0, tb1, tb2,
          res0, res1, si0, si1, si2, si3, sg0, sg1, ss0, ss1):
        qp = [[qa0, qa1, qa2], [qb0, qb1, qb2]]
        qt = [[ta0, ta1, ta2], [tb0, tb1, tb2]]
        res = [res0, res1]
        sem_i = [si0, si1, si2, si3]
        sem_g = [sg0, sg1]
        sem_s = [ss0, ss1]
        cid = lax.axis_index("c")
        sid = lax.axis_index("s")
        wid = cid * NS + sid
        pltpu.sync_copy(zero_h, acc.at[pl.ds(sid * ZR, ZR), :])
        pltpu.sync_copy(zero_h.at[pl.ds(0, UNIT), :], res0)
        pltpu.sync_copy(zero_h.at[pl.ds(0, UNIT), :], res1)
        plsc.subcore_barrier()

        base_f = wid * per_w
        iota = _iota()

        def unit_base(u):
            return jnp.minimum(base_f + u * UNIT, F - UNIT)

        def fire_idx(u, r):
            b = unit_base(u)
            for j in range(6):
                pltpu.async_copy(fidx_h.at[j, pl.ds(b, UNIT)], ib.at[r, j],
                                 sem_i[r])

        def wait_idx(u, r):
            b = unit_base(u)
            for j in range(6):
                pltpu.make_async_copy(fidx_h.at[j, pl.ds(b, UNIT)],
                                      ib.at[r, j], sem_i[r]).wait()

        def fire_gath(r, p):
            for c in range(3):
                pltpu.async_copy(table_h.at[ib.at[r, c]], qp[p][c], sem_g[p])
                pltpu.async_copy(table_h.at[ib.at[r, 3 + c]], qt[p][c],
                                 sem_g[p])

        def wait_gath(r, p):
            for c in range(3):
                pltpu.make_async_copy(table_h.at[ib.at[r, c]], qp[p][c],
                                      sem_g[p]).wait()
                pltpu.make_async_copy(table_h.at[ib.at[r, 3 + c]], qt[p][c],
                                      sem_g[p]).wait()

        def fire_scat(r, p):
            for j in range(3):
                pltpu.async_copy(res[p], acc.at[ib.at[r, j]], sem_s[p],
                                 add=True)

        def wait_scat(r, p):
            for j in range(3):
                pltpu.make_async_copy(res[p], acc.at[ib.at[r, j]],
                                      sem_s[p]).wait()

        def compute(u, p):
            sc = jnp.where(base_f + u * UNIT < F, 1.0, 0.0)
            svec = jnp.broadcast_to(sc.astype(jnp.float32), (L,))
            b0, b1, b2 = qp[p]
            c0, c1, c2 = qt[p]
            rr = res[p]
            for g in range(G):
                rows = iota + _cvec(g * L)

                def ld(ref, c):
                    return plsc.load_gather(ref, [rows, _cvec(c)])

                p0x, p0y, p0z = ld(b0, 0), ld(b0, 1), ld(b0, 2)
                p1x, p1y, p1z = ld(b1, 0), ld(b1, 1), ld(b1, 2)
                p2x, p2y, p2z = ld(b2, 0), ld(b2, 1), ld(b2, 2)
                t0u, t0v = ld(c0, 3), ld(c0, 4)
                t1u, t1v = ld(c1, 3), ld(c1, 4)
                t2u, t2v = ld(c2, 3), ld(c2, 4)
                e1x, e1y, e1z = p1x - p0x, p1y - p0y, p1z - p0z
                e2x, e2y, e2z = p2x - p0x, p2y - p0y, p2z - p0z
                nx = e1y * e2z - e1z * e2y
                ny = e1z * e2x - e1x * e2z
                nz = e1x * e2y - e1y * e2x
                u1, v1 = t1u - t0u, t1v - t0v
                u2, v2 = t2u - t0u, t2v - t0v
                den = u1 * v2 - v1 * u2
                den_safe = jnp.where(den > _fvec(0.0),
                                     jnp.maximum(den, _fvec(1e-6)),
                                     jnp.minimum(den, _fvec(-1e-6)))
                inv = svec / den_safe
                tgx = (e1x * v2 - e2x * v1) * inv
                tgy = (e1y * v2 - e2y * v1) * inv
                tgz = (e1z * v2 - e2z * v1) * inv
                plsc.store_scatter(rr, [rows, _cvec(0)], nx * svec)
                plsc.store_scatter(rr, [rows, _cvec(1)], ny * svec)
                plsc.store_scatter(rr, [rows, _cvec(2)], nz * svec)
                plsc.store_scatter(rr, [rows, _cvec(3)], tgx)
                plsc.store_scatter(rr, [rows, _cvec(4)], tgy)
                plsc.store_scatter(rr, [rows, _cvec(5)], tgz)

        for u in range(4):
            fire_idx(u, u)
        wait_idx(0, 0)
        fire_gath(0, 0)
        wait_gath(0, 0)
        wait_idx(1, 1)
        fire_gath(1, 1)
        compute(0, 0)
        fire_scat(0, 0)
        wait_gath(1, 1)
        wait_idx(2, 2)
        fire_gath(2, 0)
        compute(1, 1)
        fire_scat(1, 1)

        @pl.loop(0, K)
        def _(kk):
            for d in range(4):
                u = 2 + 4 * kk + d
                r_u = (2 + d) % 4
                r_n1 = (3 + d) % 4
                r_n2 = d % 4
                p = d % 2
                q = 1 - p
                wait_gath(r_u, p)
                wait_scat(r_n2, p)
                fire_idx(u + 2, r_n2)
                wait_idx(u + 1, r_n1)
                fire_gath(r_n1, q)
                compute(u, p)
                fire_scat(r_u, p)

        wait_gath(2, 0)
        wait_scat(0, 0)
        wait_idx(N - 1, 3)
        fire_gath(3, 1)
        compute(N - 2, 0)
        fire_scat(2, 0)
        wait_gath(3, 1)
        wait_scat(1, 1)
        compute(N - 1, 1)
        fire_scat(3, 1)
        wait_scat(2, 0)
        wait_scat(3, 1)

        plsc.subcore_barrier()
        pltpu.sync_copy(acc.at[pl.ds(sid * ZR, ZR), :],
                        out_h.at[cid, pl.ds(sid * ZR, ZR), :])

    return k


def _finalize_kernel(V, Vp):
    WV = Vp // NW
    LASTW = V - (NW - 1) * WV
    assert 0 < LASTW <= WV and LASTW % 8 == 0

    @functools.partial(
        pl.kernel,
        mesh=_MESH,
        out_type=jax.ShapeDtypeStruct((3, 2 * V), jnp.float32),
        scratch_types=[
            pltpu.VMEM((WV, ROW), jnp.float32),
            pltpu.VMEM((WV, ROW), jnp.float32),
            pltpu.VMEM((3, WV), jnp.float32),
            pltpu.VMEM((3, WV), jnp.float32),
        ],
        compiler_params=_CP,
    )
    def k(in_h, out_h, a0, a1, nob, tob):
        cid = lax.axis_index("c")
        sid = lax.axis_index("s")
        wid = cid * NS + sid
        b = wid * WV
        pltpu.sync_copy(in_h.at[0, pl.ds(b, WV), :], a0)
        pltpu.sync_copy(in_h.at[1, pl.ds(b, WV), :], a1)
        iota = _iota()

        @pl.loop(0, WV // L)
        def _(g):
            rows = iota + g * L

            def ld(c):
                cc = _cvec(c)
                return (plsc.load_gather(a0, [rows, cc]) +
                        plsc.load_gather(a1, [rows, cc]))

            nx, ny, nz = ld(0), ld(1), ld(2)
            tx, ty, tz = ld(3), ld(4), ld(5)
            d = nx * nx + ny * ny + nz * nz
            cond = d > _fvec(1e-20)
            zero = _fvec(0.0)
            nx = jnp.where(cond, nx, zero)
            ny = jnp.where(cond, ny, zero)
            nz = jnp.where(cond, nz, _fvec(1.0))
            dsel = jnp.where(cond, d, _fvec(1.0))
            r = _rsqrt(jnp.maximum(dsel, _fvec(1e-20)))
            onx, ony, onz = nx * r, ny * r, nz * r
            dt = tx * tx + ty * ty + tz * tz
            rt = _rsqrt(jnp.maximum(dt, _fvec(1e-20)))
            ttx, tty, ttz = tx * rt, ty * rt, tz * rt
            dtn = ttx * onx + tty * ony + ttz * onz
            wx = ttx - dtn * onx
            wy = tty - dtn * ony
            wz = ttz - dtn * onz
            dw = wx * wx + wy * wy + wz * wz
            rw = _rsqrt(jnp.maximum(dw, _fvec(1e-20)))
            sl = pl.ds(g * L, L)
            nob[0, sl] = onx
            nob[1, sl] = ony
            nob[2, sl] = onz
            tob[0, sl] = wx * rw
            tob[1, sl] = wy * rw
            tob[2, sl] = wz * rw

        @pl.when(wid < NW - 1)
        def _():
            for c in range(3):
                pltpu.sync_copy(nob.at[c], out_h.at[c, pl.ds(b, WV)])
                pltpu.sync_copy(tob.at[c], out_h.at[c, pl.ds(V + b, WV)])

        @pl.when(wid == NW - 1)
        def _():
            for c in range(3):
                pltpu.sync_copy(nob.at[c, pl.ds(0, LASTW)],
                                out_h.at[c, pl.ds(b, LASTW)])
                pltpu.sync_copy(tob.at[c, pl.ds(0, LASTW)],
                                out_h.at[c, pl.ds(V + b, LASTW)])

    return k


def kernel(positions, texcoords, faces, uv_faces):
    V = positions.shape[0]
    F = faces.shape[0]
    Vp = -(-V // (NW * L)) * (NW * L)
    zero = jnp.zeros((Vp // NS, ROW), jnp.float32)

    table = jnp.pad(
        jnp.concatenate([positions.astype(jnp.float32),
                         texcoords.astype(jnp.float32)], axis=1),
        ((0, 0), (0, ROW - 5)))
    fidx = jnp.concatenate([faces.astype(jnp.int32).T,
                            uv_faces.astype(jnp.int32).T], axis=0)

    partial = _accumulate_kernel(V, Vp, F)(table, fidx, zero)
    out_soa = _finalize_kernel(V, Vp)(partial)
    return out_soa.T


# speedup vs baseline: 7.6709x; 1.8892x over previous
import dataclasses
import functools

import jax
import jax.numpy as jnp
from jax import lax
from jax.experimental import pallas as pl
from jax.experimental.pallas import tpu as pltpu
from jax.experimental.pallas import tpu_sc as plsc

NC = 2
NS = 16
NW = NC * NS
L = 16
UNIT = 80
ROW = 8

_CP = pltpu.CompilerParams(use_tc_tiling_on_sc=False)
if "needs_layout_passes" in pltpu.CompilerParams.__dataclass_fields__:
    _CP = dataclasses.replace(_CP, needs_layout_passes=False)

_MESH = plsc.VectorSubcoreMesh(core_axis_name="c", subcore_axis_name="s")


def _iota():
    return lax.iota(jnp.int32, L)


def _cvec(c):
    return jnp.full((L,), c, jnp.int32)


def _fvec(x):
    return jnp.full((L,), x, jnp.float32)


def _rsqrt(x):
    i = plsc.bitcast(x, jnp.int32)
    i = jnp.full((L,), 0x5F3759DF, jnp.int32) - lax.shift_right_logical(
        i, jnp.full((L,), 1, jnp.int32))
    y = plsc.bitcast(i, jnp.float32)
    h = x * _fvec(0.5)
    for _ in range(3):
        y = y * (_fvec(1.5) - h * y * y)
    return y


def _accumulate_kernel(V, Vp, F):
    per_w = -(-F // (NW * 4 * UNIT)) * (4 * UNIT)
    N = per_w // UNIT
    assert N % 4 == 0 and N >= 8 and F % UNIT == 0
    K = (N - 4) // 4
    ZR = Vp // NS
    LS = V - (NS - 1) * ZR
    assert 0 < LS <= ZR and LS % 8 == 0
    NCH = 8
    CH = ZR // NCH
    assert CH % L == 0
    LS2 = LS - (NCH - 1) * CH
    assert 0 < LS2 <= CH and LS2 % 8 == 0 and (NCH - 1) * CH <= LS
    G = UNIT // L

    @functools.partial(
        pl.kernel,
        mesh=_MESH,
        out_type=jax.ShapeDtypeStruct((NC, Vp, ROW), jnp.float32),
        scratch_types=[
            pltpu.VMEM_SHARED((Vp, ROW), jnp.float32),
            pltpu.VMEM_SHARED((Vp, ROW), jnp.float32),
            pltpu.VMEM((5, ZR // NCH), jnp.float32),
            pltpu.VMEM((ZR // NCH, ROW), jnp.float32),
            pltpu.VMEM((4, 6, UNIT), jnp.int32),
            pltpu.VMEM((UNIT, ROW), jnp.float32),
            pltpu.VMEM((UNIT, ROW), jnp.float32),
            pltpu.VMEM((UNIT, ROW), jnp.float32),
            pltpu.VMEM((UNIT, ROW), jnp.float32),
            pltpu.VMEM((UNIT, ROW), jnp.float32),
            pltpu.VMEM((UNIT, ROW), jnp.float32),
            pltpu.VMEM((UNIT, ROW), jnp.float32),
            pltpu.VMEM((UNIT, ROW), jnp.float32),
            pltpu.VMEM((UNIT, ROW), jnp.float32),
            pltpu.VMEM((UNIT, ROW), jnp.float32),
            pltpu.VMEM((UNIT, ROW), jnp.float32),
            pltpu.VMEM((UNIT, ROW), jnp.float32),
            pltpu.VMEM((UNIT, ROW), jnp.float32),
            pltpu.VMEM((UNIT, ROW), jnp.float32),
            pltpu.SemaphoreType.DMA,
            pltpu.SemaphoreType.DMA,
            pltpu.SemaphoreType.DMA,
            pltpu.SemaphoreType.DMA,
            pltpu.SemaphoreType.DMA,
            pltpu.SemaphoreType.DMA,
            pltpu.SemaphoreType.DMA,
            pltpu.SemaphoreType.DMA,
        ],
        compiler_params=_CP,
    )
    def k(pt_h, fidx_h, zero_h, out_h, acc, table_sh, sta_in, sta_out, ib,
          qa0, qa1, qa2, qb0, qb1, qb2, ta0, ta1, ta2, tb0, tb1, tb2,
          res0, res1, si0, si1, si2, si3, sg0, sg1, ss0, ss1):
        qp = [[qa0, qa1, qa2], [qb0, qb1, qb2]]
        qt = [[ta0, ta1, ta2], [tb0, tb1, tb2]]
        res = [res0, res1]
        sem_i = [si0, si1, si2, si3]
        sem_g = [sg0, sg1]
        sem_s = [ss0, ss1]
        cid = lax.axis_index("c")
        sid = lax.axis_index("s")
        wid = cid * NS + sid
        pltpu.sync_copy(zero_h, acc.at[pl.ds(sid * ZR, ZR), :])
        pltpu.sync_copy(zero_h.at[pl.ds(0, UNIT), :], res0)
        pltpu.sync_copy(zero_h.at[pl.ds(0, UNIT), :], res1)
        iota = _iota()

        base_r = sid * ZR
        for ch in range(NCH):
            r0 = base_r + ch * CH
            if ch < NCH - 1:
                for c in range(5):
                    pltpu.sync_copy(pt_h.at[c, pl.ds(r0, CH)], sta_in.at[c])
            else:
                @pl.when(sid < NS - 1)
                def _():
                    for c in range(5):
                        pltpu.sync_copy(pt_h.at[c, pl.ds(r0, CH)],
                                        sta_in.at[c])

                @pl.when(sid == NS - 1)
                def _():
                    for c in range(5):
                        pltpu.sync_copy(pt_h.at[c, pl.ds(r0, LS2)],
                                        sta_in.at[c, pl.ds(0, LS2)])

            @pl.loop(0, CH // L)
            def _(g):
                rows = iota + g * L
                for c in range(5):
                    plsc.store_scatter(sta_out, [rows, _cvec(c)],
                                       sta_in[c, pl.ds(g * L, L)])

            pltpu.sync_copy(sta_out, table_sh.at[pl.ds(r0, CH), :])
        plsc.subcore_barrier()

        base_f = wid * per_w

        def unit_base(u):
            return jnp.minimum(base_f + u * UNIT, F - UNIT)

        def fire_idx(u, r):
            b = unit_base(u)
            for j in range(6):
                pltpu.async_copy(fidx_h.at[j, pl.ds(b, UNIT)], ib.at[r, j],
                                 sem_i[r])

        def wait_idx(u, r):
            b = unit_base(u)
            for j in range(6):
                pltpu.make_async_copy(fidx_h.at[j, pl.ds(b, UNIT)],
                                      ib.at[r, j], sem_i[r]).wait()

        def fire_gath(r, p):
            for c in range(3):
                pltpu.async_copy(table_sh.at[ib.at[r, c]], qp[p][c], sem_g[p])
                pltpu.async_copy(table_sh.at[ib.at[r, 3 + c]], qt[p][c],
                                 sem_g[p])

        def wait_gath(r, p):
            for c in range(3):
                pltpu.make_async_copy(table_sh.at[ib.at[r, c]], qp[p][c],
                                      sem_g[p]).wait()
                pltpu.make_async_copy(table_sh.at[ib.at[r, 3 + c]], qt[p][c],
                                      sem_g[p]).wait()

        def fire_scat(r, p):
            for j in range(3):
                pltpu.async_copy(res[p], acc.at[ib.at[r, j]], sem_s[p],
                                 add=True)

        def wait_scat(r, p):
            for j in range(3):
                pltpu.make_async_copy(res[p], acc.at[ib.at[r, j]],
                                      sem_s[p]).wait()

        def compute(u, p):
            sc = jnp.where(base_f + u * UNIT < F, 1.0, 0.0)
            svec = jnp.broadcast_to(sc.astype(jnp.float32), (L,))
            b0, b1, b2 = qp[p]
            c0, c1, c2 = qt[p]
            rr = res[p]
            for g in range(G):
                rows = iota + _cvec(g * L)

                def ld(ref, c):
                    return plsc.load_gather(ref, [rows, _cvec(c)])

                p0x, p0y, p0z = ld(b0, 0), ld(b0, 1), ld(b0, 2)
                p1x, p1y, p1z = ld(b1, 0), ld(b1, 1), ld(b1, 2)
                p2x, p2y, p2z = ld(b2, 0), ld(b2, 1), ld(b2, 2)
                t0u, t0v = ld(c0, 3), ld(c0, 4)
                t1u, t1v = ld(c1, 3), ld(c1, 4)
                t2u, t2v = ld(c2, 3), ld(c2, 4)
                e1x, e1y, e1z = p1x - p0x, p1y - p0y, p1z - p0z
                e2x, e2y, e2z = p2x - p0x, p2y - p0y, p2z - p0z
                nx = e1y * e2z - e1z * e2y
                ny = e1z * e2x - e1x * e2z
                nz = e1x * e2y - e1y * e2x
                u1, v1 = t1u - t0u, t1v - t0v
                u2, v2 = t2u - t0u, t2v - t0v
                den = u1 * v2 - v1 * u2
                den_safe = jnp.where(den > _fvec(0.0),
                                     jnp.maximum(den, _fvec(1e-6)),
                                     jnp.minimum(den, _fvec(-1e-6)))
                inv = svec / den_safe
                tgx = (e1x * v2 - e2x * v1) * inv
                tgy = (e1y * v2 - e2y * v1) * inv
                tgz = (e1z * v2 - e2z * v1) * inv
                plsc.store_scatter(rr, [rows, _cvec(0)], nx * svec)
                plsc.store_scatter(rr, [rows, _cvec(1)], ny * svec)
                plsc.store_scatter(rr, [rows, _cvec(2)], nz * svec)
                plsc.store_scatter(rr, [rows, _cvec(3)], tgx)
                plsc.store_scatter(rr, [rows, _cvec(4)], tgy)
                plsc.store_scatter(rr, [rows, _cvec(5)], tgz)

        for u in range(4):
            fire_idx(u, u)
        wait_idx(0, 0)
        fire_gath(0, 0)
        wait_gath(0, 0)
        wait_idx(1, 1)
        fire_gath(1, 1)
        compute(0, 0)
        fire_scat(0, 0)
        wait_gath(1, 1)
        wait_idx(2, 2)
        fire_gath(2, 0)
        compute(1, 1)
        fire_scat(1, 1)

        @pl.loop(0, K)
        def _(kk):
            for d in range(4):
                u = 2 + 4 * kk + d
                r_u = (2 + d) % 4
                r_n1 = (3 + d) % 4
                r_n2 = d % 4
                p = d % 2
                q = 1 - p
                wait_gath(r_u, p)
                wait_scat(r_n2, p)
                fire_idx(u + 2, r_n2)
                wait_idx(u + 1, r_n1)
                fire_gath(r_n1, q)
                compute(u, p)
                fire_scat(r_u, p)

        wait_gath(2, 0)
        wait_scat(0, 0)
        wait_idx(N - 1, 3)
        fire_gath(3, 1)
        compute(N - 2, 0)
        fire_scat(2, 0)
        wait_gath(3, 1)
        wait_scat(1, 1)
        compute(N - 1, 1)
        fire_scat(3, 1)
        wait_scat(2, 0)
        wait_scat(3, 1)

        plsc.subcore_barrier()
        pltpu.sync_copy(acc.at[pl.ds(sid * ZR, ZR), :],
                        out_h.at[cid, pl.ds(sid * ZR, ZR), :])

    return k


def _finalize_kernel(V, Vp):
    WV = Vp // NW
    LASTW = V - (NW - 1) * WV
    assert 0 < LASTW <= WV and LASTW % 8 == 0

    @functools.partial(
        pl.kernel,
        mesh=_MESH,
        out_type=jax.ShapeDtypeStruct((3, 2 * V), jnp.float32),
        scratch_types=[
            pltpu.VMEM((WV, ROW), jnp.float32),
            pltpu.VMEM((WV, ROW), jnp.float32),
            pltpu.VMEM((3, WV), jnp.float32),
            pltpu.VMEM((3, WV), jnp.float32),
        ],
        compiler_params=_CP,
    )
    def k(in_h, out_h, a0, a1, nob, tob):
        cid = lax.axis_index("c")
        sid = lax.axis_index("s")
        wid = cid * NS + sid
        b = wid * WV
        pltpu.sync_copy(in_h.at[0, pl.ds(b, WV), :], a0)
        pltpu.sync_copy(in_h.at[1, pl.ds(b, WV), :], a1)
        iota = _iota()

        @pl.loop(0, WV // L)
        def _(g):
            rows = iota + g * L

            def ld(c):
                cc = _cvec(c)
                return (plsc.load_gather(a0, [rows, cc]) +
                        plsc.load_gather(a1, [rows, cc]))

            nx, ny, nz = ld(0), ld(1), ld(2)
            tx, ty, tz = ld(3), ld(4), ld(5)
            d = nx * nx + ny * ny + nz * nz
            cond = d > _fvec(1e-20)
            zero = _fvec(0.0)
            nx = jnp.where(cond, nx, zero)
            ny = jnp.where(cond, ny, zero)
            nz = jnp.where(cond, nz, _fvec(1.0))
            dsel = jnp.where(cond, d, _fvec(1.0))
            r = _rsqrt(jnp.maximum(dsel, _fvec(1e-20)))
            onx, ony, onz = nx * r, ny * r, nz * r
            dt = tx * tx + ty * ty + tz * tz
            rt = _rsqrt(jnp.maximum(dt, _fvec(1e-20)))
            ttx, tty, ttz = tx * rt, ty * rt, tz * rt
            dtn = ttx * onx + tty * ony + ttz * onz
            wx = ttx - dtn * onx
            wy = tty - dtn * ony
            wz = ttz - dtn * onz
            dw = wx * wx + wy * wy + wz * wz
            rw = _rsqrt(jnp.maximum(dw, _fvec(1e-20)))
            sl = pl.ds(g * L, L)
            nob[0, sl] = onx
            nob[1, sl] = ony
            nob[2, sl] = onz
            tob[0, sl] = wx * rw
            tob[1, sl] = wy * rw
            tob[2, sl] = wz * rw

        @pl.when(wid < NW - 1)
        def _():
            for c in range(3):
                pltpu.sync_copy(nob.at[c], out_h.at[c, pl.ds(b, WV)])
                pltpu.sync_copy(tob.at[c], out_h.at[c, pl.ds(V + b, WV)])

        @pl.when(wid == NW - 1)
        def _():
            for c in range(3):
                pltpu.sync_copy(nob.at[c, pl.ds(0, LASTW)],
                                out_h.at[c, pl.ds(b, LASTW)])
                pltpu.sync_copy(tob.at[c, pl.ds(0, LASTW)],
                                out_h.at[c, pl.ds(V + b, LASTW)])

    return k


def kernel(positions, texcoords, faces, uv_faces):
    V = positions.shape[0]
    F = faces.shape[0]
    Vp = -(-V // (NW * L)) * (NW * L)
    zero = jnp.zeros((Vp // NS, ROW), jnp.float32)

    pt_soa = jnp.concatenate([positions.astype(jnp.float32).T,
                              texcoords.astype(jnp.float32).T], axis=0)
    fidx = jnp.concatenate([faces.astype(jnp.int32).T,
                            uv_faces.astype(jnp.int32).T], axis=0)

    partial = _accumulate_kernel(V, Vp, F)(pt_soa, fidx, zero)
    out_soa = _finalize_kernel(V, Vp)(partial)
    return out_soa.T


# speedup vs baseline: 7.8280x; 1.0205x over previous
import dataclasses
import functools

import jax
import jax.numpy as jnp
from jax import lax
from jax.experimental import pallas as pl
from jax.experimental.pallas import tpu as pltpu
from jax.experimental.pallas import tpu_sc as plsc

NC = 2
NS = 16
NW = NC * NS
L = 16
UNIT = 80
ROW = 8

_CP = pltpu.CompilerParams(use_tc_tiling_on_sc=False)
if "needs_layout_passes" in pltpu.CompilerParams.__dataclass_fields__:
    _CP = dataclasses.replace(_CP, needs_layout_passes=False)

_MESH = plsc.VectorSubcoreMesh(core_axis_name="c", subcore_axis_name="s")


def _iota():
    return lax.iota(jnp.int32, L)


def _cvec(c):
    return jnp.full((L,), c, jnp.int32)


def _fvec(x):
    return jnp.full((L,), x, jnp.float32)


def _rsqrt(x):
    i = plsc.bitcast(x, jnp.int32)
    i = jnp.full((L,), 0x5F3759DF, jnp.int32) - lax.shift_right_logical(
        i, jnp.full((L,), 1, jnp.int32))
    y = plsc.bitcast(i, jnp.float32)
    h = x * _fvec(0.5)
    for _ in range(2):
        y = y * (_fvec(1.5) - h * y * y)
    return y


def _accumulate_kernel(V, Vp, F):
    per_w = -(-F // (NW * 4 * UNIT)) * (4 * UNIT)
    N = per_w // UNIT
    assert N % 4 == 0 and N >= 8 and F % UNIT == 0
    K = (N - 4) // 4
    ZR = Vp // NS
    LS = V - (NS - 1) * ZR
    assert 0 < LS <= ZR and LS % 8 == 0
    NCH = 8
    CH = ZR // NCH
    assert CH % L == 0
    LS2 = LS - (NCH - 1) * CH
    assert 0 < LS2 <= CH and LS2 % 8 == 0 and (NCH - 1) * CH <= LS
    G = UNIT // L

    @functools.partial(
        pl.kernel,
        mesh=_MESH,
        out_type=jax.ShapeDtypeStruct((NC, Vp, ROW), jnp.float32),
        scratch_types=[
            pltpu.VMEM_SHARED((Vp, ROW), jnp.float32),
            pltpu.VMEM_SHARED((Vp, ROW), jnp.float32),
            pltpu.VMEM((5, ZR // NCH), jnp.float32),
            pltpu.VMEM((ZR // NCH, ROW), jnp.float32),
            pltpu.VMEM((4, 6, UNIT), jnp.int32),
            pltpu.VMEM((UNIT, ROW), jnp.float32),
            pltpu.VMEM((UNIT, ROW), jnp.float32),
            pltpu.VMEM((UNIT, ROW), jnp.float32),
            pltpu.VMEM((UNIT, ROW), jnp.float32),
            pltpu.VMEM((UNIT, ROW), jnp.float32),
            pltpu.VMEM((UNIT, ROW), jnp.float32),
            pltpu.VMEM((UNIT, ROW), jnp.float32),
            pltpu.VMEM((UNIT, ROW), jnp.float32),
            pltpu.VMEM((UNIT, ROW), jnp.float32),
            pltpu.VMEM((UNIT, ROW), jnp.float32),
            pltpu.VMEM((UNIT, ROW), jnp.float32),
            pltpu.VMEM((UNIT, ROW), jnp.float32),
            pltpu.VMEM((UNIT, ROW), jnp.float32),
            pltpu.VMEM((UNIT, ROW), jnp.float32),
            pltpu.SemaphoreType.DMA,
            pltpu.SemaphoreType.DMA,
            pltpu.SemaphoreType.DMA,
            pltpu.SemaphoreType.DMA,
            pltpu.SemaphoreType.DMA,
            pltpu.SemaphoreType.DMA,
            pltpu.SemaphoreType.DMA,
            pltpu.SemaphoreType.DMA,
        ],
        compiler_params=_CP,
    )
    def k(pt_h, fidx_h, zero_h, out_h, acc, table_sh, sta_in, sta_out, ib,
          qa0, qa1, qa2, qb0, qb1, qb2, ta0, ta1, ta2, tb0, tb1, tb2,
          res0, res1, si0, si1, si2, si3, sg0, sg1, ss0, ss1):
        qp = [[qa0, qa1, qa2], [qb0, qb1, qb2]]
        qt = [[ta0, ta1, ta2], [tb0, tb1, tb2]]
        res = [res0, res1]
        sem_i = [si0, si1, si2, si3]
        sem_g = [sg0, sg1]
        sem_s = [ss0, ss1]
        cid = lax.axis_index("c")
        sid = lax.axis_index("s")
        wid = cid * NS + sid
        pltpu.sync_copy(zero_h, acc.at[pl.ds(sid * ZR, ZR), :])
        pltpu.sync_copy(zero_h.at[pl.ds(0, UNIT), :], res0)
        pltpu.sync_copy(zero_h.at[pl.ds(0, UNIT), :], res1)
        iota = _iota()

        base_f = wid * per_w

        def unit_base(u):
            return jnp.minimum(base_f + u * UNIT, F - UNIT)

        def fire_idx(u, r):
            b = unit_base(u)
            pltpu.async_copy(fidx_h.at[:, pl.ds(b, UNIT)], ib.at[r],
                             sem_i[r])

        def wait_idx(u, r):
            b = unit_base(u)
            pltpu.make_async_copy(fidx_h.at[:, pl.ds(b, UNIT)],
                                  ib.at[r], sem_i[r]).wait()

        for u in range(4):
            fire_idx(u, u)

        base_r = sid * ZR
        for ch in range(NCH):
            r0 = base_r + ch * CH
            if ch < NCH - 1:
                for c in range(5):
                    pltpu.sync_copy(pt_h.at[c, pl.ds(r0, CH)], sta_in.at[c])
            else:
                @pl.when(sid < NS - 1)
                def _():
                    for c in range(5):
                        pltpu.sync_copy(pt_h.at[c, pl.ds(r0, CH)],
                                        sta_in.at[c])

                @pl.when(sid == NS - 1)
                def _():
                    for c in range(5):
                        pltpu.sync_copy(pt_h.at[c, pl.ds(r0, LS2)],
                                        sta_in.at[c, pl.ds(0, LS2)])

            @pl.loop(0, CH // L)
            def _(g):
                rows = iota + g * L
                for c in range(5):
                    plsc.store_scatter(sta_out, [rows, _cvec(c)],
                                       sta_in[c, pl.ds(g * L, L)])

            pltpu.sync_copy(sta_out, table_sh.at[pl.ds(r0, CH), :])
        plsc.subcore_barrier()

        def fire_gath(r, p):
            for c in range(3):
                pltpu.async_copy(table_sh.at[ib.at[r, c]], qp[p][c], sem_g[p])
                pltpu.async_copy(table_sh.at[ib.at[r, 3 + c]], qt[p][c],
                                 sem_g[p])

        def wait_gath(r, p):
            for c in range(3):
                pltpu.make_async_copy(table_sh.at[ib.at[r, c]], qp[p][c],
                                      sem_g[p]).wait()
                pltpu.make_async_copy(table_sh.at[ib.at[r, 3 + c]], qt[p][c],
                                      sem_g[p]).wait()

        def fire_scat(r, p):
            for j in range(3):
                pltpu.async_copy(res[p], acc.at[ib.at[r, j]], sem_s[p],
                                 add=True)

        def wait_scat(r, p):
            for j in range(3):
                pltpu.make_async_copy(res[p], acc.at[ib.at[r, j]],
                                      sem_s[p]).wait()

        def compute(u, p):
            sc = jnp.where(base_f + u * UNIT < F, 1.0, 0.0)
            svec = jnp.broadcast_to(sc.astype(jnp.float32), (L,))
            b0, b1, b2 = qp[p]
            c0, c1, c2 = qt[p]
            rr = res[p]
            for g in range(G):
                rows = iota + _cvec(g * L)

                def ld(ref, c):
                    return plsc.load_gather(ref, [rows, _cvec(c)])

                p0x, p0y, p0z = ld(b0, 0), ld(b0, 1), ld(b0, 2)
                p1x, p1y, p1z = ld(b1, 0), ld(b1, 1), ld(b1, 2)
                p2x, p2y, p2z = ld(b2, 0), ld(b2, 1), ld(b2, 2)
                t0u, t0v = ld(c0, 3), ld(c0, 4)
                t1u, t1v = ld(c1, 3), ld(c1, 4)
                t2u, t2v = ld(c2, 3), ld(c2, 4)
                e1x, e1y, e1z = p1x - p0x, p1y - p0y, p1z - p0z
                e2x, e2y, e2z = p2x - p0x, p2y - p0y, p2z - p0z
                nx = e1y * e2z - e1z * e2y
                ny = e1z * e2x - e1x * e2z
                nz = e1x * e2y - e1y * e2x
                u1, v1 = t1u - t0u, t1v - t0v
                u2, v2 = t2u - t0u, t2v - t0v
                den = u1 * v2 - v1 * u2
                den_safe = jnp.where(den > _fvec(0.0),
                                     jnp.maximum(den, _fvec(1e-6)),
                                     jnp.minimum(den, _fvec(-1e-6)))
                inv = svec / den_safe
                tgx = (e1x * v2 - e2x * v1) * inv
                tgy = (e1y * v2 - e2y * v1) * inv
                tgz = (e1z * v2 - e2z * v1) * inv
                plsc.store_scatter(rr, [rows, _cvec(0)], nx * svec)
                plsc.store_scatter(rr, [rows, _cvec(1)], ny * svec)
                plsc.store_scatter(rr, [rows, _cvec(2)], nz * svec)
                plsc.store_scatter(rr, [rows, _cvec(3)], tgx)
                plsc.store_scatter(rr, [rows, _cvec(4)], tgy)
                plsc.store_scatter(rr, [rows, _cvec(5)], tgz)

        wait_idx(0, 0)
        fire_gath(0, 0)
        wait_gath(0, 0)
        wait_idx(1, 1)
        fire_gath(1, 1)
        compute(0, 0)
        fire_scat(0, 0)
        wait_gath(1, 1)
        wait_idx(2, 2)
        fire_gath(2, 0)
        compute(1, 1)
        fire_scat(1, 1)

        @pl.loop(0, K)
        def _(kk):
            for d in range(4):
                u = 2 + 4 * kk + d
                r_u = (2 + d) % 4
                r_n1 = (3 + d) % 4
                r_n2 = d % 4
                p = d % 2
                q = 1 - p
                wait_gath(r_u, p)
                wait_scat(r_n2, p)
                fire_idx(u + 2, r_n2)
                wait_idx(u + 1, r_n1)
                fire_gath(r_n1, q)
                compute(u, p)
                fire_scat(r_u, p)

        wait_gath(2, 0)
        wait_scat(0, 0)
        wait_idx(N - 1, 3)
        fire_gath(3, 1)
        compute(N - 2, 0)
        fire_scat(2, 0)
        wait_gath(3, 1)
        wait_scat(1, 1)
        compute(N - 1, 1)
        fire_scat(3, 1)
        wait_scat(2, 0)
        wait_scat(3, 1)

        plsc.subcore_barrier()
        pltpu.sync_copy(acc.at[pl.ds(sid * ZR, ZR), :],
                        out_h.at[cid, pl.ds(sid * ZR, ZR), :])

    return k


def _finalize_kernel(V, Vp):
    WV = Vp // NW
    LASTW = V - (NW - 1) * WV
    assert 0 < LASTW <= WV and LASTW % 8 == 0

    @functools.partial(
        pl.kernel,
        mesh=_MESH,
        out_type=jax.ShapeDtypeStruct((3, 2 * V), jnp.float32),
        scratch_types=[
            pltpu.VMEM((WV, ROW), jnp.float32),
            pltpu.VMEM((WV, ROW), jnp.float32),
            pltpu.VMEM((3, WV), jnp.float32),
            pltpu.VMEM((3, WV), jnp.float32),
        ],
        compiler_params=_CP,
    )
    def k(in_h, out_h, a0, a1, nob, tob):
        cid = lax.axis_index("c")
        sid = lax.axis_index("s")
        wid = cid * NS + sid
        b = wid * WV
        pltpu.sync_copy(in_h.at[0, pl.ds(b, WV), :], a0)
        pltpu.sync_copy(in_h.at[1, pl.ds(b, WV), :], a1)
        iota = _iota()

        @pl.loop(0, WV // L)
        def _(g):
            rows = iota + g * L

            def ld(c):
                cc = _cvec(c)
                return (plsc.load_gather(a0, [rows, cc]) +
                        plsc.load_gather(a1, [rows, cc]))

            nx, ny, nz = ld(0), ld(1), ld(2)
            tx, ty, tz = ld(3), ld(4), ld(5)
            d = nx * nx + ny * ny + nz * nz
            cond = d > _fvec(1e-20)
            zero = _fvec(0.0)
            nx = jnp.where(cond, nx, zero)
            ny = jnp.where(cond, ny, zero)
            nz = jnp.where(cond, nz, _fvec(1.0))
            dsel = jnp.where(cond, d, _fvec(1.0))
            r = _rsqrt(jnp.maximum(dsel, _fvec(1e-20)))
            onx, ony, onz = nx * r, ny * r, nz * r
            dt = tx * tx + ty * ty + tz * tz
            rt = _rsqrt(jnp.maximum(dt, _fvec(1e-20)))
            ttx, tty, ttz = tx * rt, ty * rt, tz * rt
            dtn = ttx * onx + tty * ony + ttz * onz
            wx = ttx - dtn * onx
            wy = tty - dtn * ony
            wz = ttz - dtn * onz
            dw = wx * wx + wy * wy + wz * wz
            rw = _rsqrt(jnp.maximum(dw, _fvec(1e-20)))
            sl = pl.ds(g * L, L)
            nob[0, sl] = onx
            nob[1, sl] = ony
            nob[2, sl] = onz
            tob[0, sl] = wx * rw
            tob[1, sl] = wy * rw
            tob[2, sl] = wz * rw

        @pl.when(wid < NW - 1)
        def _():
            for c in range(3):
                pltpu.sync_copy(nob.at[c], out_h.at[c, pl.ds(b, WV)])
                pltpu.sync_copy(tob.at[c], out_h.at[c, pl.ds(V + b, WV)])

        @pl.when(wid == NW - 1)
        def _():
            for c in range(3):
                pltpu.sync_copy(nob.at[c, pl.ds(0, LASTW)],
                                out_h.at[c, pl.ds(b, LASTW)])
                pltpu.sync_copy(tob.at[c, pl.ds(0, LASTW)],
                                out_h.at[c, pl.ds(V + b, LASTW)])

    return k


def kernel(positions, texcoords, faces, uv_faces):
    V = positions.shape[0]
    F = faces.shape[0]
    Vp = -(-V // (NW * L)) * (NW * L)
    zero = jnp.zeros((Vp // NS, ROW), jnp.float32)

    pt_soa = jnp.concatenate([positions.astype(jnp.float32).T,
                              texcoords.astype(jnp.float32).T], axis=0)
    fidx = jnp.concatenate([faces.astype(jnp.int32).T,
                            uv_faces.astype(jnp.int32).T], axis=0)

    partial = _accumulate_kernel(V, Vp, F)(pt_soa, fidx, zero)
    out_soa = _finalize_kernel(V, Vp)(partial)
    return out_soa.T


# speedup vs baseline: 7.9997x; 1.0219x over previous
import dataclasses
import functools

import jax
import jax.numpy as jnp
from jax import lax
from jax.experimental import pallas as pl
from jax.experimental.pallas import tpu as pltpu
from jax.experimental.pallas import tpu_sc as plsc

NC = 2
NS = 16
NW = NC * NS
L = 16
UNIT = 80
ROW = 8

_CP = pltpu.CompilerParams(use_tc_tiling_on_sc=False)
if "needs_layout_passes" in pltpu.CompilerParams.__dataclass_fields__:
    _CP = dataclasses.replace(_CP, needs_layout_passes=False)

_MESH = plsc.VectorSubcoreMesh(core_axis_name="c", subcore_axis_name="s")


def _iota():
    return lax.iota(jnp.int32, L)


def _cvec(c):
    return jnp.full((L,), c, jnp.int32)


def _fvec(x):
    return jnp.full((L,), x, jnp.float32)


def _rsqrt(x):
    i = plsc.bitcast(x, jnp.int32)
    i = jnp.full((L,), 0x5F3759DF, jnp.int32) - lax.shift_right_logical(
        i, jnp.full((L,), 1, jnp.int32))
    y = plsc.bitcast(i, jnp.float32)
    h = x * _fvec(0.5)
    for _ in range(3):
        y = y * (_fvec(1.5) - h * y * y)
    return y


def _accumulate_kernel(V, Vp, F):
    per_w = -(-F // (NW * 4 * UNIT)) * (4 * UNIT)
    N = per_w // UNIT
    assert N % 4 == 0 and N >= 8 and F % UNIT == 0
    K = (N - 4) // 4
    ZR = Vp // NS
    LS = V - (NS - 1) * ZR
    assert 0 < LS <= ZR and LS % 8 == 0
    NCH = 8
    CH = ZR // NCH
    assert CH % L == 0
    LS2 = LS - (NCH - 1) * CH
    assert 0 < LS2 <= CH and LS2 % 8 == 0 and (NCH - 1) * CH <= LS
    G = UNIT // L

    @functools.partial(
        pl.kernel,
        mesh=_MESH,
        out_type=jax.ShapeDtypeStruct((NC, Vp, ROW), jnp.float32),
        scratch_types=[
            pltpu.VMEM_SHARED((Vp, ROW), jnp.float32),
            pltpu.VMEM_SHARED((Vp, ROW), jnp.float32),
            pltpu.VMEM((5, ZR // NCH), jnp.float32),
            pltpu.VMEM((ZR // NCH, ROW), jnp.float32),
            pltpu.VMEM((4, 6, UNIT), jnp.int32),
            pltpu.VMEM((UNIT, ROW), jnp.float32),
            pltpu.VMEM((UNIT, ROW), jnp.float32),
            pltpu.VMEM((UNIT, ROW), jnp.float32),
            pltpu.VMEM((UNIT, ROW), jnp.float32),
            pltpu.VMEM((UNIT, ROW), jnp.float32),
            pltpu.VMEM((UNIT, ROW), jnp.float32),
            pltpu.VMEM((UNIT, ROW), jnp.float32),
            pltpu.VMEM((UNIT, ROW), jnp.float32),
            pltpu.VMEM((UNIT, ROW), jnp.float32),
            pltpu.VMEM((UNIT, ROW), jnp.float32),
            pltpu.VMEM((UNIT, ROW), jnp.float32),
            pltpu.VMEM((UNIT, ROW), jnp.float32),
            pltpu.VMEM((UNIT, ROW), jnp.float32),
            pltpu.VMEM((UNIT, ROW), jnp.float32),
            pltpu.SemaphoreType.DMA,
            pltpu.SemaphoreType.DMA,
            pltpu.SemaphoreType.DMA,
            pltpu.SemaphoreType.DMA,
            pltpu.SemaphoreType.DMA,
            pltpu.SemaphoreType.DMA,
            pltpu.SemaphoreType.DMA,
            pltpu.SemaphoreType.DMA,
        ],
        compiler_params=_CP,
    )
    def k(pt_h, fidx_h, out_h, acc, table_sh, sta_in, sta_out, ib,
          qa0, qa1, qa2, qb0, qb1, qb2, ta0, ta1, ta2, tb0, tb1, tb2,
          res0, res1, si0, si1, si2, si3, sg0, sg1, ss0, ss1):
        qp = [[qa0, qa1, qa2], [qb0, qb1, qb2]]
        qt = [[ta0, ta1, ta2], [tb0, tb1, tb2]]
        res = [res0, res1]
        sem_i = [si0, si1, si2, si3]
        sem_g = [sg0, sg1]
        sem_s = [ss0, ss1]
        cid = lax.axis_index("c")
        sid = lax.axis_index("s")
        wid = cid * NS + sid
        iota = _iota()
        zvec = _fvec(0.0)

        for g in range(G):
            zrows = iota + _cvec(g * L)
            for c in (6, 7):
                plsc.store_scatter(res0, [zrows, _cvec(c)], zvec)
                plsc.store_scatter(res1, [zrows, _cvec(c)], zvec)

        @pl.loop(0, CH // L)
        def _(g):
            zrows = iota + g * L
            for c in range(ROW):
                plsc.store_scatter(sta_out, [zrows, _cvec(c)], zvec)

        base_f = wid * per_w

        def unit_base(u):
            return jnp.minimum(base_f + u * UNIT, F - UNIT)

        def fire_idx(u, r):
            b = unit_base(u)
            pltpu.async_copy(fidx_h.at[:, pl.ds(b, UNIT)], ib.at[r],
                             sem_i[r])

        def wait_idx(u, r):
            b = unit_base(u)
            pltpu.make_async_copy(fidx_h.at[:, pl.ds(b, UNIT)],
                                  ib.at[r], sem_i[r]).wait()

        for u in range(4):
            fire_idx(u, u)

        base_r = sid * ZR
        for ch in range(NCH):
            pltpu.sync_copy(sta_out, acc.at[pl.ds(base_r + ch * CH, CH), :])

        for ch in range(NCH):
            r0 = base_r + ch * CH
            if ch < NCH - 1:
                for c in range(5):
                    pltpu.sync_copy(pt_h.at[c, pl.ds(r0, CH)], sta_in.at[c])
            else:
                @pl.when(sid < NS - 1)
                def _():
                    for c in range(5):
                        pltpu.sync_copy(pt_h.at[c, pl.ds(r0, CH)],
                                        sta_in.at[c])

                @pl.when(sid == NS - 1)
                def _():
                    for c in range(5):
                        pltpu.sync_copy(pt_h.at[c, pl.ds(r0, LS2)],
                                        sta_in.at[c, pl.ds(0, LS2)])

            @pl.loop(0, CH // L)
            def _(g):
                rows = iota + g * L
                for c in range(5):
                    plsc.store_scatter(sta_out, [rows, _cvec(c)],
                                       sta_in[c, pl.ds(g * L, L)])

            pltpu.sync_copy(sta_out, table_sh.at[pl.ds(r0, CH), :])
        plsc.subcore_barrier()

        def fire_gath(r, p):
            for c in range(3):
                pltpu.async_copy(table_sh.at[ib.at[r, c]], qp[p][c], sem_g[p])
                pltpu.async_copy(table_sh.at[ib.at[r, 3 + c]], qt[p][c],
                                 sem_g[p])

        def wait_gath(r, p):
            for c in range(3):
                pltpu.make_async_copy(table_sh.at[ib.at[r, c]], qp[p][c],
                                      sem_g[p]).wait()
                pltpu.make_async_copy(table_sh.at[ib.at[r, 3 + c]], qt[p][c],
                                      sem_g[p]).wait()

        def fire_scat(r, p):
            for j in range(3):
                pltpu.async_copy(res[p], acc.at[ib.at[r, j]], sem_s[p],
                                 add=True)

        def wait_scat(r, p):
            for j in range(3):
                pltpu.make_async_copy(res[p], acc.at[ib.at[r, j]],
                                      sem_s[p]).wait()

        def compute(u, p):
            sc = jnp.where(base_f + u * UNIT < F, 1.0, 0.0)
            svec = jnp.broadcast_to(sc.astype(jnp.float32), (L,))
            b0, b1, b2 = qp[p]
            c0, c1, c2 = qt[p]
            rr = res[p]
            for g in range(G):
                rows = iota + _cvec(g * L)

                def ld(ref, c):
                    return plsc.load_gather(ref, [rows, _cvec(c)])

                p0x, p0y, p0z = ld(b0, 0), ld(b0, 1), ld(b0, 2)
                p1x, p1y, p1z = ld(b1, 0), ld(b1, 1), ld(b1, 2)
                p2x, p2y, p2z = ld(b2, 0), ld(b2, 1), ld(b2, 2)
                t0u, t0v = ld(c0, 3), ld(c0, 4)
                t1u, t1v = ld(c1, 3), ld(c1, 4)
                t2u, t2v = ld(c2, 3), ld(c2, 4)
                e1x, e1y, e1z = p1x - p0x, p1y - p0y, p1z - p0z
                e2x, e2y, e2z = p2x - p0x, p2y - p0y, p2z - p0z
                nx = e1y * e2z - e1z * e2y
                ny = e1z * e2x - e1x * e2z
                nz = e1x * e2y - e1y * e2x
                u1, v1 = t1u - t0u, t1v - t0v
                u2, v2 = t2u - t0u, t2v - t0v
                den = u1 * v2 - v1 * u2
                den_safe = jnp.where(den > _fvec(0.0),
                                     jnp.maximum(den, _fvec(1e-6)),
                                     jnp.minimum(den, _fvec(-1e-6)))
                inv = svec / den_safe
                tgx = (e1x * v2 - e2x * v1) * inv
                tgy = (e1y * v2 - e2y * v1) * inv
                tgz = (e1z * v2 - e2z * v1) * inv
                plsc.store_scatter(rr, [rows, _cvec(0)], nx * svec)
                plsc.store_scatter(rr, [rows, _cvec(1)], ny * svec)
                plsc.store_scatter(rr, [rows, _cvec(2)], nz * svec)
                plsc.store_scatter(rr, [rows, _cvec(3)], tgx)
                plsc.store_scatter(rr, [rows, _cvec(4)], tgy)
                plsc.store_scatter(rr, [rows, _cvec(5)], tgz)

        wait_idx(0, 0)
        fire_gath(0, 0)
        wait_gath(0, 0)
        wait_idx(1, 1)
        fire_gath(1, 1)
        compute(0, 0)
        fire_scat(0, 0)
        wait_gath(1, 1)
        wait_idx(2, 2)
        fire_gath(2, 0)
        compute(1, 1)
        fire_scat(1, 1)

        @pl.loop(0, K)
        def _(kk):
            for d in range(4):
                u = 2 + 4 * kk + d
                r_u = (2 + d) % 4
                r_n1 = (3 + d) % 4
                r_n2 = d % 4
                p = d % 2
                q = 1 - p
                wait_gath(r_u, p)
                wait_scat(r_n2, p)
                fire_idx(u + 2, r_n2)
                wait_idx(u + 1, r_n1)
                fire_gath(r_n1, q)
                compute(u, p)
                fire_scat(r_u, p)

        wait_gath(2, 0)
        wait_scat(0, 0)
        wait_idx(N - 1, 3)
        fire_gath(3, 1)
        compute(N - 2, 0)
        fire_scat(2, 0)
        wait_gath(3, 1)
        wait_scat(1, 1)
        compute(N - 1, 1)
        fire_scat(3, 1)
        wait_scat(2, 0)
        wait_scat(3, 1)

        plsc.subcore_barrier()
        pltpu.sync_copy(acc.at[pl.ds(sid * ZR, ZR), :],
                        out_h.at[cid, pl.ds(sid * ZR, ZR), :])

    return k


def _finalize_kernel(V, Vp):
    WV = Vp // NW
    LASTW = V - (NW - 1) * WV
    assert 0 < LASTW <= WV and LASTW % 8 == 0

    @functools.partial(
        pl.kernel,
        mesh=_MESH,
        out_type=jax.ShapeDtypeStruct((3, 2 * V), jnp.float32),
        scratch_types=[
            pltpu.VMEM((WV, ROW), jnp.float32),
            pltpu.VMEM((WV, ROW), jnp.float32),
            pltpu.VMEM((3, WV), jnp.float32),
            pltpu.VMEM((3, WV), jnp.float32),
        ],
        compiler_params=_CP,
    )
    def k(in_h, out_h, a0, a1, nob, tob):
        cid = lax.axis_index("c")
        sid = lax.axis_index("s")
        wid = cid * NS + sid
        b = wid * WV
        pltpu.sync_copy(in_h.at[0, pl.ds(b, WV), :], a0)
        pltpu.sync_copy(in_h.at[1, pl.ds(b, WV), :], a1)
        iota = _iota()

        @pl.loop(0, WV // L)
        def _(g):
            rows = iota + g * L

            def ld(c):
                cc = _cvec(c)
                return (plsc.load_gather(a0, [rows, cc]) +
                        plsc.load_gather(a1, [rows, cc]))

            nx, ny, nz = ld(0), ld(1), ld(2)
            tx, ty, tz = ld(3), ld(4), ld(5)
            d = nx * nx + ny * ny + nz * nz
            cond = d > _fvec(1e-20)
            zero = _fvec(0.0)
            nx = jnp.where(cond, nx, zero)
            ny = jnp.where(cond, ny, zero)
            nz = jnp.where(cond, nz, _fvec(1.0))
            dsel = jnp.where(cond, d, _fvec(1.0))
            r = _rsqrt(jnp.maximum(dsel, _fvec(1e-20)))
            onx, ony, onz = nx * r, ny * r, nz * r
            dt = tx * tx + ty * ty + tz * tz
            rt = _rsqrt(jnp.maximum(dt, _fvec(1e-20)))
            ttx, tty, ttz = tx * rt, ty * rt, tz * rt
            dtn = ttx * onx + tty * ony + ttz * onz
            wx = ttx - dtn * onx
            wy = tty - dtn * ony
            wz = ttz - dtn * onz
            dw = wx * wx + wy * wy + wz * wz
            rw = _rsqrt(jnp.maximum(dw, _fvec(1e-20)))
            sl = pl.ds(g * L, L)
            nob[0, sl] = onx
            nob[1, sl] = ony
            nob[2, sl] = onz
            tob[0, sl] = wx * rw
            tob[1, sl] = wy * rw
            tob[2, sl] = wz * rw

        @pl.when(wid < NW - 1)
        def _():
            for c in range(3):
                pltpu.sync_copy(nob.at[c], out_h.at[c, pl.ds(b, WV)])
                pltpu.sync_copy(tob.at[c], out_h.at[c, pl.ds(V + b, WV)])

        @pl.when(wid == NW - 1)
        def _():
            for c in range(3):
                pltpu.sync_copy(nob.at[c, pl.ds(0, LASTW)],
                                out_h.at[c, pl.ds(b, LASTW)])
                pltpu.sync_copy(tob.at[c, pl.ds(0, LASTW)],
                                out_h.at[c, pl.ds(V + b, LASTW)])

    return k


def kernel(positions, texcoords, faces, uv_faces):
    V = positions.shape[0]
    F = faces.shape[0]
    Vp = -(-V // (NW * L)) * (NW * L)
    pt_soa = jnp.concatenate([positions.astype(jnp.float32).T,
                              texcoords.astype(jnp.float32).T], axis=0)
    fidx = jnp.concatenate([faces.astype(jnp.int32).T,
                            uv_faces.astype(jnp.int32).T], axis=0)

    partial = _accumulate_kernel(V, Vp, F)(pt_soa, fidx)
    out_soa = _finalize_kernel(V, Vp)(partial)
    return out_soa.T
